# baseline (speedup 1.0000x reference)
import functools

import jax
import jax.numpy as jnp
from jax.experimental import pallas as pl
from jax.experimental.pallas import tpu as pltpu

_CAP = 64
_LOSS_COEF = 0.01
_SB = 256


def _routing_body(x_ref, w_ref, idx1_ref, idx2_ref, g1_ref, g2_ref,
                  cnt_ref, gsum_ref):
    b = pl.program_id(1)
    x = x_ref[0]
    w = w_ref[...]
    sb = x.shape[0]
    e = w.shape[1]

    logits = jnp.dot(x, w, preferred_element_type=jnp.float32)
    mx = jnp.max(logits, axis=-1, keepdims=True)
    ex = jnp.exp(logits - mx)
    raw = ex / jnp.sum(ex, axis=-1, keepdims=True)

    eidx = jax.lax.broadcasted_iota(jnp.int32, (sb, e), 1)
    m1 = jnp.max(raw, axis=-1, keepdims=True)
    idx1 = jnp.min(jnp.where(raw == m1, eidx, e), axis=-1, keepdims=True)
    oh1 = (eidx == idx1).astype(jnp.float32)
    gate1 = jnp.sum(raw * oh1, axis=-1, keepdims=True)

    raw2 = raw * (1.0 - oh1)
    m2 = jnp.max(raw2, axis=-1, keepdims=True)
    idx2 = jnp.min(jnp.where(raw2 == m2, eidx, e), axis=-1, keepdims=True)
    oh2 = (eidx == idx2).astype(jnp.float32)
    gate2 = jnp.sum(raw * oh2, axis=-1, keepdims=True)

    denom = gate1 + gate2 + 1e-9
    idx1_ref[0] = idx1
    idx2_ref[0] = idx2
    g1_ref[0] = gate1 / denom
    g2_ref[0] = gate2 / denom

    csum = jnp.sum(oh1, axis=0, keepdims=True)
    gsum = jnp.sum(raw, axis=0, keepdims=True)

    @pl.when(b == 0)
    def _init():
        cnt_ref[0] = csum
        gsum_ref[0] = gsum

    @pl.when(b != 0)
    def _acc():
        cnt_ref[0] += csum
        gsum_ref[0] += gsum


def _emit_body(aux_scale, idx1_ref, idx2_ref, g1_ref, g2_ref, cnt_ref,
               cnt_all_ref, gsum_all_ref, comb_ref, disp_ref, aux_ref,
               c1_scr, c2_scr):
    b = pl.program_id(1)

    @pl.when(b == 0)
    def _reset():
        c1_scr[...] = jnp.zeros_like(c1_scr)
        c2_scr[...] = jnp.zeros_like(c2_scr)

    idx1 = idx1_ref[0]
    idx2 = idx2_ref[0]
    g1 = g1_ref[0]
    g2 = g2_ref[0]
    sb = idx1.shape[0]
    e = c1_scr.shape[1]

    eidx = jax.lax.broadcasted_iota(jnp.int32, (sb, e), 1)
    oh1 = (eidx == idx1).astype(jnp.float32)
    oh2 = (eidx == idx2).astype(jnp.float32)

    r = jax.lax.broadcasted_iota(jnp.int32, (sb, sb), 0)
    c = jax.lax.broadcasted_iota(jnp.int32, (sb, sb), 1)
    tril = (r >= c).astype(jnp.float32)
    cum1 = jnp.dot(tril, oh1, preferred_element_type=jnp.float32)
    cum2 = jnp.dot(tril, oh2, preferred_element_type=jnp.float32)

    c1pre = c1_scr[...]
    c2pre = c2_scr[...]
    cnt1 = jnp.minimum(cnt_ref[0], float(_CAP))

    pos1 = cum1 - 1.0 + c1pre
    p1 = jnp.sum(pos1 * oh1, axis=-1, keepdims=True)
    keep1 = (p1 < _CAP).astype(jnp.float32)
    pos2 = cum2 - 1.0 + c2pre + cnt1
    p2 = jnp.sum(pos2 * oh2, axis=-1, keepdims=True)
    keep2 = (p2 < _CAP).astype(jnp.float32)

    c1_scr[...] = c1pre + cum1[sb - 1:sb, :]
    c2_scr[...] = c2pre + cum2[sb - 1:sb, :]

    cidx = jax.lax.broadcasted_iota(jnp.int32, (sb, _CAP), 1)
    ohc1 = (cidx == p1.astype(jnp.int32)).astype(jnp.float32)
    ohc2 = (cidx == p2.astype(jnp.int32)).astype(jnp.float32)

    t1 = (g1 * keep1) * oh1
    t2 = (g2 * keep2) * oh2
    comb = (t1[:, :, None] * ohc1[:, None, :]
            + t2[:, :, None] * ohc2[:, None, :])
    comb_ref[0] = comb
    disp_ref[0] = (comb > 0.0).astype(jnp.float32)

    prod = gsum_all_ref[...] * cnt_all_ref[...]
    aux_ref[...] = jnp.sum(prod, axis=(0, 2), keepdims=True)[0] * aux_scale


def _moe_gating(inputs, gating_weight):
    g, s, m = inputs.shape
    e = gating_weight.shape[1]
    nb = s // _SB

    tok_shape = (g * nb, _SB, 1)
    routing = pl.pallas_call(
        _routing_body,
        grid=(g, nb),
        in_specs=[
            pl.BlockSpec((1, _SB, m), lambda gi, bi: (gi, bi, 0)),
            pl.BlockSpec((m, e), lambda gi, bi: (0, 0)),
        ],
        out_specs=[
            pl.BlockSpec((1, _SB, 1), lambda gi, bi, nb=nb: (gi * nb + bi, 0, 0)),
            pl.BlockSpec((1, _SB, 1), lambda gi, bi, nb=nb: (gi * nb + bi, 0, 0)),
            pl.BlockSpec((1, _SB, 1), lambda gi, bi, nb=nb: (gi * nb + bi, 0, 0)),
            pl.BlockSpec((1, _SB, 1), lambda gi, bi, nb=nb: (gi * nb + bi, 0, 0)),
            pl.BlockSpec((1, 1, e), lambda gi, bi: (gi, 0, 0)),
            pl.BlockSpec((1, 1, e), lambda gi, bi: (gi, 0, 0)),
        ],
        out_shape=[
            jax.ShapeDtypeStruct(tok_shape, jnp.int32),
            jax.ShapeDtypeStruct(tok_shape, jnp.int32),
            jax.ShapeDtypeStruct(tok_shape, jnp.float32),
            jax.ShapeDtypeStruct(tok_shape, jnp.float32),
            jax.ShapeDtypeStruct((g, 1, e), jnp.float32),
            jax.ShapeDtypeStruct((g, 1, e), jnp.float32),
        ],
        compiler_params=pltpu.CompilerParams(
            dimension_semantics=("parallel", "arbitrary")),
    )
    idx1, idx2, g1n, g2n, cnt, gsum = routing(inputs, gating_weight)

    aux_scale = _LOSS_COEF * e / (g * s * s)
    emit = pl.pallas_call(
        functools.partial(_emit_body, aux_scale),
        grid=(g, nb),
        in_specs=[
            pl.BlockSpec((1, _SB, 1), lambda gi, bi, nb=nb: (gi * nb + bi, 0, 0)),
            pl.BlockSpec((1, _SB, 1), lambda gi, bi, nb=nb: (gi * nb + bi, 0, 0)),
            pl.BlockSpec((1, _SB, 1), lambda gi, bi, nb=nb: (gi * nb + bi, 0, 0)),
            pl.BlockSpec((1, _SB, 1), lambda gi, bi, nb=nb: (gi * nb + bi, 0, 0)),
            pl.BlockSpec((1, 1, e), lambda gi, bi: (gi, 0, 0)),
            pl.BlockSpec((g, 1, e), lambda gi, bi: (0, 0, 0)),
            pl.BlockSpec((g, 1, e), lambda gi, bi: (0, 0, 0)),
        ],
        out_specs=[
            pl.BlockSpec((1, _SB, e, _CAP), lambda gi, bi: (gi, bi, 0, 0)),
            pl.BlockSpec((1, _SB, e, _CAP), lambda gi, bi: (gi, bi, 0, 0)),
            pl.BlockSpec((1, 1), lambda gi, bi: (0, 0)),
        ],
        out_shape=[
            jax.ShapeDtypeStruct((g, s, e, _CAP), jnp.float32),
            jax.ShapeDtypeStruct((g, s, e, _CAP), jnp.float32),
            jax.ShapeDtypeStruct((1, 1), jnp.float32),
        ],
        scratch_shapes=[
            pltpu.VMEM((1, e), jnp.float32),
            pltpu.VMEM((1, e), jnp.float32),
        ],
        compiler_params=pltpu.CompilerParams(
            dimension_semantics=("parallel", "arbitrary")),
    )
    comb, disp, aux = emit(idx1, idx2, g1n, g2n, cnt, cnt, gsum)
    return comb, disp, aux[0, 0]


def kernel(inputs, gating_weight, total_token_num):
    del total_token_num
    return _moe_gating(inputs, gating_weight)


# speedup vs baseline: 4.0150x; 4.0150x over previous
import functools

import jax
import jax.numpy as jnp
from jax.experimental import pallas as pl
from jax.experimental.pallas import tpu as pltpu

_CAP = 64
_LOSS_COEF = 0.01
_SB = 256


def _routing_body(x_ref, w_ref, idx1_ref, idx2_ref, g1_ref, g2_ref,
                  cnt_ref, gsum_ref):
    b = pl.program_id(1)
    x = x_ref[0]
    w = w_ref[...]
    sb = x.shape[0]
    e = w.shape[1]

    logits = jnp.dot(x, w, preferred_element_type=jnp.float32)
    mx = jnp.max(logits, axis=-1, keepdims=True)
    ex = jnp.exp(logits - mx)
    raw = ex / jnp.sum(ex, axis=-1, keepdims=True)

    eidx = jax.lax.broadcasted_iota(jnp.int32, (sb, e), 1)
    m1 = jnp.max(raw, axis=-1, keepdims=True)
    idx1 = jnp.min(jnp.where(raw == m1, eidx, e), axis=-1, keepdims=True)
    oh1 = (eidx == idx1).astype(jnp.float32)
    gate1 = jnp.sum(raw * oh1, axis=-1, keepdims=True)

    raw2 = raw * (1.0 - oh1)
    m2 = jnp.max(raw2, axis=-1, keepdims=True)
    idx2 = jnp.min(jnp.where(raw2 == m2, eidx, e), axis=-1, keepdims=True)
    oh2 = (eidx == idx2).astype(jnp.float32)
    gate2 = jnp.sum(raw * oh2, axis=-1, keepdims=True)

    denom = gate1 + gate2 + 1e-9
    idx1_ref[0] = idx1.T
    idx2_ref[0] = idx2.T
    g1_ref[0] = (gate1 / denom).T
    g2_ref[0] = (gate2 / denom).T

    csum = jnp.sum(oh1, axis=0, keepdims=True)
    gsum = jnp.sum(raw, axis=0, keepdims=True)

    @pl.when(b == 0)
    def _init():
        cnt_ref[0] = csum
        gsum_ref[0] = gsum

    @pl.when(b != 0)
    def _acc():
        cnt_ref[0] += csum
        gsum_ref[0] += gsum


def _emit_body(aux_scale, idx1_ref, idx2_ref, g1_ref, g2_ref, cnt_ref,
               cnt_all_ref, gsum_all_ref, comb_ref, disp_ref, aux_ref,
               c1_scr, c2_scr):
    b = pl.program_id(1)

    @pl.when(b == 0)
    def _reset():
        c1_scr[...] = jnp.zeros_like(c1_scr)
        c2_scr[...] = jnp.zeros_like(c2_scr)

    idx1 = idx1_ref[0]
    idx2 = idx2_ref[0]
    g1 = g1_ref[0]
    g2 = g2_ref[0]
    sb = idx1.shape[1]
    e = c1_scr.shape[0]

    eidx = jax.lax.broadcasted_iota(jnp.int32, (e, sb), 0)
    oh1 = (eidx == idx1).astype(jnp.float32)
    oh2 = (eidx == idx2).astype(jnp.float32)

    r = jax.lax.broadcasted_iota(jnp.int32, (sb, sb), 0)
    c = jax.lax.broadcasted_iota(jnp.int32, (sb, sb), 1)
    triu = (r <= c).astype(jnp.float32)
    cum1 = jnp.dot(oh1, triu, preferred_element_type=jnp.float32)
    cum2 = jnp.dot(oh2, triu, preferred_element_type=jnp.float32)

    c1pre = c1_scr[...]
    c2pre = c2_scr[...]
    cnt1 = jnp.minimum(cnt_ref[0].T, float(_CAP))

    pos1 = cum1 - 1.0 + c1pre
    p1 = jnp.sum(pos1 * oh1, axis=0, keepdims=True)
    keep1 = (p1 < _CAP).astype(jnp.float32)
    pos2 = cum2 - 1.0 + c2pre + cnt1
    p2 = jnp.sum(pos2 * oh2, axis=0, keepdims=True)
    keep2 = (p2 < _CAP).astype(jnp.float32)

    c1_scr[...] = c1pre + cum1[:, sb - 1:sb]
    c2_scr[...] = c2pre + cum2[:, sb - 1:sb]

    cidx = jax.lax.broadcasted_iota(jnp.int32, (_CAP, sb), 0)
    ohc1 = (cidx == p1.astype(jnp.int32)).astype(jnp.float32)
    ohc2 = (cidx == p2.astype(jnp.int32)).astype(jnp.float32)

    t1 = (g1 * keep1) * oh1
    t2 = (g2 * keep2) * oh2
    comb = (t1[:, None, :] * ohc1[None, :, :]
            + t2[:, None, :] * ohc2[None, :, :])
    comb_ref[0] = comb
    disp_ref[0] = (comb > 0.0).astype(jnp.float32)

    prod = gsum_all_ref[...] * cnt_all_ref[...]
    aux_ref[...] = jnp.sum(prod, axis=(0, 2), keepdims=True)[0] * aux_scale


def _moe_gating(inputs, gating_weight):
    g, s, m = inputs.shape
    e = gating_weight.shape[1]
    nb = s // _SB

    tok_shape = (g * nb, 1, _SB)
    routing = pl.pallas_call(
        _routing_body,
        grid=(g, nb),
        in_specs=[
            pl.BlockSpec((1, _SB, m), lambda gi, bi: (gi, bi, 0)),
            pl.BlockSpec((m, e), lambda gi, bi: (0, 0)),
        ],
        out_specs=[
            pl.BlockSpec((1, 1, _SB), lambda gi, bi, nb=nb: (gi * nb + bi, 0, 0)),
            pl.BlockSpec((1, 1, _SB), lambda gi, bi, nb=nb: (gi * nb + bi, 0, 0)),
            pl.BlockSpec((1, 1, _SB), lambda gi, bi, nb=nb: (gi * nb + bi, 0, 0)),
            pl.BlockSpec((1, 1, _SB), lambda gi, bi, nb=nb: (gi * nb + bi, 0, 0)),
            pl.BlockSpec((1, 1, e), lambda gi, bi: (gi, 0, 0)),
            pl.BlockSpec((1, 1, e), lambda gi, bi: (gi, 0, 0)),
        ],
        out_shape=[
            jax.ShapeDtypeStruct(tok_shape, jnp.int32),
            jax.ShapeDtypeStruct(tok_shape, jnp.int32),
            jax.ShapeDtypeStruct(tok_shape, jnp.float32),
            jax.ShapeDtypeStruct(tok_shape, jnp.float32),
            jax.ShapeDtypeStruct((g, 1, e), jnp.float32),
            jax.ShapeDtypeStruct((g, 1, e), jnp.float32),
        ],
        compiler_params=pltpu.CompilerParams(
            dimension_semantics=("parallel", "arbitrary")),
    )
    idx1, idx2, g1n, g2n, cnt, gsum = routing(inputs, gating_weight)

    aux_scale = _LOSS_COEF * e / (g * s * s)
    emit = pl.pallas_call(
        functools.partial(_emit_body, aux_scale),
        grid=(g, nb),
        in_specs=[
            pl.BlockSpec((1, 1, _SB), lambda gi, bi, nb=nb: (gi * nb + bi, 0, 0)),
            pl.BlockSpec((1, 1, _SB), lambda gi, bi, nb=nb: (gi * nb + bi, 0, 0)),
            pl.BlockSpec((1, 1, _SB), lambda gi, bi, nb=nb: (gi * nb + bi, 0, 0)),
            pl.BlockSpec((1, 1, _SB), lambda gi, bi, nb=nb: (gi * nb + bi, 0, 0)),
            pl.BlockSpec((1, 1, e), lambda gi, bi: (gi, 0, 0)),
            pl.BlockSpec((g, 1, e), lambda gi, bi: (0, 0, 0)),
            pl.BlockSpec((g, 1, e), lambda gi, bi: (0, 0, 0)),
        ],
        out_specs=[
            pl.BlockSpec((1, e, _CAP, _SB), lambda gi, bi: (gi, 0, 0, bi)),
            pl.BlockSpec((1, e, _CAP, _SB), lambda gi, bi: (gi, 0, 0, bi)),
            pl.BlockSpec((1, 1), lambda gi, bi: (0, 0)),
        ],
        out_shape=[
            jax.ShapeDtypeStruct((g, e, _CAP, s), jnp.float32),
            jax.ShapeDtypeStruct((g, e, _CAP, s), jnp.float32),
            jax.ShapeDtypeStruct((1, 1), jnp.float32),
        ],
        scratch_shapes=[
            pltpu.VMEM((e, 1), jnp.float32),
            pltpu.VMEM((e, 1), jnp.float32),
        ],
        compiler_params=pltpu.CompilerParams(
            dimension_semantics=("parallel", "arbitrary")),
    )
    combT, dispT, aux = emit(idx1, idx2, g1n, g2n, cnt, cnt, gsum)
    comb = jnp.transpose(combT, (0, 3, 1, 2))
    disp = jnp.transpose(dispT, (0, 3, 1, 2))
    return comb, disp, aux[0, 0]


def kernel(inputs, gating_weight, total_token_num):
    del total_token_num
    return _moe_gating(inputs, gating_weight)


# speedup vs baseline: 4.3749x; 1.0897x over previous
import functools

import jax
import jax.numpy as jnp
from jax.experimental import pallas as pl
from jax.experimental.pallas import tpu as pltpu

_CAP = 64
_LOSS_COEF = 0.01
_SB = 512


def _routing_body(x_ref, w_ref, idx1_ref, idx2_ref, g1_ref, g2_ref,
                  cnt_ref, gsum_ref):
    b = pl.program_id(1)
    x = x_ref[0]
    w = w_ref[...]
    sb = x.shape[0]
    e = w.shape[1]

    logits = jnp.dot(x, w, preferred_element_type=jnp.float32)
    mx = jnp.max(logits, axis=-1, keepdims=True)
    ex = jnp.exp(logits - mx)
    raw = ex / jnp.sum(ex, axis=-1, keepdims=True)

    eidx = jax.lax.broadcasted_iota(jnp.int32, (sb, e), 1)
    m1 = jnp.max(raw, axis=-1, keepdims=True)
    idx1 = jnp.min(jnp.where(raw == m1, eidx, e), axis=-1, keepdims=True)
    oh1 = (eidx == idx1).astype(jnp.float32)
    gate1 = jnp.sum(raw * oh1, axis=-1, keepdims=True)

    raw2 = raw * (1.0 - oh1)
    m2 = jnp.max(raw2, axis=-1, keepdims=True)
    idx2 = jnp.min(jnp.where(raw2 == m2, eidx, e), axis=-1, keepdims=True)
    oh2 = (eidx == idx2).astype(jnp.float32)
    gate2 = jnp.sum(raw * oh2, axis=-1, keepdims=True)

    denom = gate1 + gate2 + 1e-9
    idx1_ref[0] = idx1.T
    idx2_ref[0] = idx2.T
    g1_ref[0] = (gate1 / denom).T
    g2_ref[0] = (gate2 / denom).T

    csum = jnp.sum(oh1, axis=0, keepdims=True)
    gsum = jnp.sum(raw, axis=0, keepdims=True)

    @pl.when(b == 0)
    def _init():
        cnt_ref[0] = csum
        gsum_ref[0] = gsum

    @pl.when(b != 0)
    def _acc():
        cnt_ref[0] += csum
        gsum_ref[0] += gsum


def _emit_body(aux_scale, idx1_ref, idx2_ref, g1_ref, g2_ref, cnt_ref,
               cnt_all_ref, gsum_all_ref, comb_ref, disp_ref, aux_ref,
               c1_scr, c2_scr):
    b = pl.program_id(1)

    @pl.when(b == 0)
    def _reset():
        c1_scr[...] = jnp.zeros_like(c1_scr)
        c2_scr[...] = jnp.zeros_like(c2_scr)

    idx1 = idx1_ref[0]
    idx2 = idx2_ref[0]
    g1 = g1_ref[0]
    g2 = g2_ref[0]
    sb = idx1.shape[1]
    e = c1_scr.shape[0]

    eidx = jax.lax.broadcasted_iota(jnp.int32, (e, sb), 0)
    oh1 = (eidx == idx1).astype(jnp.float32)
    oh2 = (eidx == idx2).astype(jnp.float32)

    r = jax.lax.broadcasted_iota(jnp.int32, (sb, sb), 0)
    c = jax.lax.broadcasted_iota(jnp.int32, (sb, sb), 1)
    triu = (r <= c).astype(jnp.float32)
    cum1 = jnp.dot(oh1, triu, preferred_element_type=jnp.float32)
    cum2 = jnp.dot(oh2, triu, preferred_element_type=jnp.float32)

    c1pre = c1_scr[...]
    c2pre = c2_scr[...]
    cnt1 = jnp.minimum(cnt_ref[0].T, float(_CAP))

    pos1 = cum1 - 1.0 + c1pre
    p1 = jnp.sum(pos1 * oh1, axis=0, keepdims=True)
    keep1 = (p1 < _CAP).astype(jnp.float32)
    pos2 = cum2 - 1.0 + c2pre + cnt1
    p2 = jnp.sum(pos2 * oh2, axis=0, keepdims=True)
    keep2 = (p2 < _CAP).astype(jnp.float32)

    c1_scr[...] = c1pre + cum1[:, sb - 1:sb]
    c2_scr[...] = c2pre + cum2[:, sb - 1:sb]

    cidx = jax.lax.broadcasted_iota(jnp.int32, (_CAP, sb), 0)
    ohc1 = (cidx == p1.astype(jnp.int32)).astype(jnp.float32)
    ohc2 = (cidx == p2.astype(jnp.int32)).astype(jnp.float32)

    t1 = (g1 * keep1) * oh1
    t2 = (g2 * keep2) * oh2
    comb = (t1[:, None, :] * ohc1[None, :, :]
            + t2[:, None, :] * ohc2[None, :, :])
    comb_ref[0] = comb
    disp_ref[0] = (comb > 0.0).astype(jnp.float32)

    prod = gsum_all_ref[...] * cnt_all_ref[...]
    aux_ref[...] = jnp.sum(prod, axis=(0, 2), keepdims=True)[0] * aux_scale


def _moe_gating(inputs, gating_weight):
    g, s, m = inputs.shape
    e = gating_weight.shape[1]
    nb = s // _SB

    tok_shape = (g * nb, 1, _SB)
    routing = pl.pallas_call(
        _routing_body,
        grid=(g, nb),
        in_specs=[
            pl.BlockSpec((1, _SB, m), lambda gi, bi: (gi, bi, 0)),
            pl.BlockSpec((m, e), lambda gi, bi: (0, 0)),
        ],
        out_specs=[
            pl.BlockSpec((1, 1, _SB), lambda gi, bi, nb=nb: (gi * nb + bi, 0, 0)),
            pl.BlockSpec((1, 1, _SB), lambda gi, bi, nb=nb: (gi * nb + bi, 0, 0)),
            pl.BlockSpec((1, 1, _SB), lambda gi, bi, nb=nb: (gi * nb + bi, 0, 0)),
            pl.BlockSpec((1, 1, _SB), lambda gi, bi, nb=nb: (gi * nb + bi, 0, 0)),
            pl.BlockSpec((1, 1, e), lambda gi, bi: (gi, 0, 0)),
            pl.BlockSpec((1, 1, e), lambda gi, bi: (gi, 0, 0)),
        ],
        out_shape=[
            jax.ShapeDtypeStruct(tok_shape, jnp.int32),
            jax.ShapeDtypeStruct(tok_shape, jnp.int32),
            jax.ShapeDtypeStruct(tok_shape, jnp.float32),
            jax.ShapeDtypeStruct(tok_shape, jnp.float32),
            jax.ShapeDtypeStruct((g, 1, e), jnp.float32),
            jax.ShapeDtypeStruct((g, 1, e), jnp.float32),
        ],
        compiler_params=pltpu.CompilerParams(
            dimension_semantics=("parallel", "arbitrary")),
    )
    idx1, idx2, g1n, g2n, cnt, gsum = routing(inputs, gating_weight)

    aux_scale = _LOSS_COEF * e / (g * s * s)
    emit = pl.pallas_call(
        functools.partial(_emit_body, aux_scale),
        grid=(g, nb),
        in_specs=[
            pl.BlockSpec((1, 1, _SB), lambda gi, bi, nb=nb: (gi * nb + bi, 0, 0)),
            pl.BlockSpec((1, 1, _SB), lambda gi, bi, nb=nb: (gi * nb + bi, 0, 0)),
            pl.BlockSpec((1, 1, _SB), lambda gi, bi, nb=nb: (gi * nb + bi, 0, 0)),
            pl.BlockSpec((1, 1, _SB), lambda gi, bi, nb=nb: (gi * nb + bi, 0, 0)),
            pl.BlockSpec((1, 1, e), lambda gi, bi: (gi, 0, 0)),
            pl.BlockSpec((g, 1, e), lambda gi, bi: (0, 0, 0)),
            pl.BlockSpec((g, 1, e), lambda gi, bi: (0, 0, 0)),
        ],
        out_specs=[
            pl.BlockSpec((1, e, _CAP, _SB), lambda gi, bi: (gi, 0, 0, bi)),
            pl.BlockSpec((1, e, _CAP, _SB), lambda gi, bi: (gi, 0, 0, bi)),
            pl.BlockSpec((1, 1), lambda gi, bi: (0, 0)),
        ],
        out_shape=[
            jax.ShapeDtypeStruct((g, e, _CAP, s), jnp.float32),
            jax.ShapeDtypeStruct((g, e, _CAP, s), jnp.float32),
            jax.ShapeDtypeStruct((1, 1), jnp.float32),
        ],
        scratch_shapes=[
            pltpu.VMEM((e, 1), jnp.float32),
            pltpu.VMEM((e, 1), jnp.float32),
        ],
        compiler_params=pltpu.CompilerParams(
            dimension_semantics=("parallel", "arbitrary")),
    )
    combT, dispT, aux = emit(idx1, idx2, g1n, g2n, cnt, cnt, gsum)
    comb = jnp.transpose(combT, (0, 3, 1, 2))
    disp = jnp.transpose(dispT, (0, 3, 1, 2))
    return comb, disp, aux[0, 0]


def kernel(inputs, gating_weight, total_token_num):
    del total_token_num
    return _moe_gating(inputs, gating_weight)
